# baseline (speedup 1.0000x reference)
import jax
import jax.numpy as jnp
from jax import lax
from jax.experimental import pallas as pl
from jax.experimental.pallas import tpu as pltpu
from jax.experimental.pallas import tpu_sc as plsc

_D = 32
_L = 128
_K = 10
_NC = 2
_NS = 16
_NW = _NC * _NS


def _gather_body(table_hbm, idx_hbm, out_hbm, idx_all, rows0, rows1,
                 gsem0, gsem1, osem):
    wid = lax.axis_index("s") * _NC + lax.axis_index("c")
    n_vecs = idx_hbm.shape[0]
    per_w = n_vecs // _NW
    n_chunks = per_w // _K
    base = wid * per_w

    pltpu.sync_copy(idx_hbm.at[pl.ds(base, per_w)], idx_all)

    rows = (rows0, rows1)
    gsem = (gsem0, gsem1)

    def fire_gather(c, b):
        for j in range(_K):
            pltpu.async_copy(
                table_hbm.at[idx_all.at[c * _K + j]],
                rows[b].at[pl.ds(j * _L, _L)], gsem[b])

    def drain_gather(c, b):
        for j in range(_K):
            pltpu.make_async_copy(
                table_hbm.at[idx_all.at[c * _K + j]],
                rows[b].at[pl.ds(j * _L, _L)], gsem[b]).wait()

    def fire_store(c, b):
        pltpu.async_copy(
            rows[b], out_hbm.at[pl.ds((base + c * _K) * _L, _K * _L)], osem)

    def drain_store(c, b):
        pltpu.make_async_copy(
            rows[b], out_hbm.at[pl.ds((base + c * _K) * _L, _K * _L)],
            osem).wait()

    fire_gather(0, 0)
    fire_gather(1, 1)

    def pair(g, carry):
        for b in range(2):
            c = 2 * g + b
            drain_gather(c, b)
            fire_store(c, b)
            drain_store(c, b)
            fire_gather(c + 2, b)
        return carry

    lax.fori_loop(0, n_chunks // 2 - 1, pair, 0)

    for b in range(2):
        c = n_chunks - 2 + b
        drain_gather(c, b)
        fire_store(c, b)
        drain_store(c, b)


@jax.jit
def kernel(input, index):
    idx_flat = index.reshape(-1).astype(jnp.int32)
    n = idx_flat.shape[0]
    idx2d = idx_flat.reshape(n // _L, _L)
    mesh = plsc.VectorSubcoreMesh(core_axis_name="c", subcore_axis_name="s")
    per_w = (n // _L) // _NW
    out = pl.kernel(
        _gather_body,
        mesh=mesh,
        out_type=jax.ShapeDtypeStruct((n, _D), jnp.float32),
        scratch_types=[
            pltpu.VMEM((per_w, _L), jnp.int32),
            pltpu.VMEM((_K * _L, _D), jnp.float32),
            pltpu.VMEM((_K * _L, _D), jnp.float32),
            pltpu.SemaphoreType.DMA,
            pltpu.SemaphoreType.DMA,
            pltpu.SemaphoreType.DMA,
        ],
        compiler_params=pltpu.CompilerParams(use_tc_tiling_on_sc=False),
    )(input, idx2d)
    return out.reshape(index.shape + (_D,))


# speedup vs baseline: 1.3980x; 1.3980x over previous
import jax
import jax.numpy as jnp
from jax import lax
from jax.experimental import pallas as pl
from jax.experimental.pallas import tpu as pltpu
from jax.experimental.pallas import tpu_sc as plsc

_D = 32
_G = 128
_NI = 16384
_NJ = 50
_NW = 32
_SI = _NI // _NW
_C = 256
_NCHUNK = _NJ * (_SI // _C)


def _body(table4, idx_hbm, out3, idxblk, gidx0, gidx1, obuf0, obuf1,
          gath0, gath1, ost0, ost1, gsem0, gsem1, osem0, osem1):
    wid = lax.axis_index("s") * 2 + lax.axis_index("c")
    i0 = wid * _SI

    pltpu.sync_copy(idx_hbm.at[pl.ds(i0 * _NJ, _SI * _NJ)], idxblk)

    gidx = (gidx0, gidx1)
    obuf = (obuf0, obuf1)
    gath = (gath0, gath1)
    ost = (ost0, ost1)
    gsem = (gsem0, gsem1)
    osem = (osem0, osem1)

    riota = lax.iota(jnp.int32, 16)

    def prep_and_fire(k, b):
        j = k // 2
        h = k % 2
        jvec = jnp.full((16,), j, dtype=jnp.int32)
        for m in range(_C // 16):
            ivec = riota + (h * _C + 16 * m)
            raw = plsc.load_gather(idxblk, [ivec * _NJ + jvec])
            gidx[b][pl.ds(16 * m, 16)] = lax.shift_right_logical(raw, 2)
            obuf[b][pl.ds(16 * m, 16)] = (raw & 3) * _D
        for q in range(_C // 128):
            pltpu.async_copy(
                table4.at[gidx[b].at[pl.ds(q * 128, 128)]],
                gath[b].at[pl.ds(q * 128, 128)], gsem[b])

    def drain_gather(b):
        for q in range(_C // 128):
            pltpu.make_async_copy(
                table4.at[gidx[b].at[pl.ds(q * 128, 128)]],
                gath[b].at[pl.ds(q * 128, 128)], gsem[b]).wait()

    def store_dma(k, b):
        j = k // 2
        h = k % 2
        return pltpu.make_async_copy(
            ost[b], out3.at[j, :, pl.ds(i0 + h * _C, _C)], osem[b])

    def extract(b):
        for m in range(_C // 16):
            pvec = riota + 16 * m
            ovec = obuf[b][pl.ds(16 * m, 16)]
            for c in range(_D):
                val = plsc.load_gather(gath[b], [pvec, ovec + c])
                ost[b][c, pl.ds(16 * m, 16)] = val

    prep_and_fire(0, 0)

    def step(g, carry):
        for b in range(2):
            k = 2 * g + b
            nb = 1 - b

            @pl.when(k + 1 < _NCHUNK)
            def _():
                prep_and_fire(k + 1, nb)

            drain_gather(b)

            @pl.when(k >= 2)
            def _():
                store_dma(k - 2, b).wait()

            extract(b)

            pltpu.async_copy(
                ost[b], out3.at[k // 2, :, pl.ds(i0 + (k % 2) * _C, _C)],
                osem[b])
        return carry

    lax.fori_loop(0, _NCHUNK // 2, step, 0)

    store_dma(_NCHUNK - 2, 0).wait()
    store_dma(_NCHUNK - 1, 1).wait()


@jax.jit
def kernel(input, index):
    table4 = input.reshape(input.shape[0] * _D // _G, _G)
    idx = index.astype(jnp.int32).reshape(-1)
    mesh = plsc.VectorSubcoreMesh(core_axis_name="c", subcore_axis_name="s")
    out3 = pl.kernel(
        _body,
        mesh=mesh,
        out_type=jax.ShapeDtypeStruct((_NJ, _D, _NI), jnp.float32),
        scratch_types=[
            pltpu.VMEM((_SI * _NJ,), jnp.int32),
            pltpu.VMEM((_C,), jnp.int32),
            pltpu.VMEM((_C,), jnp.int32),
            pltpu.VMEM((_C,), jnp.int32),
            pltpu.VMEM((_C,), jnp.int32),
            pltpu.VMEM((_C, _G), jnp.float32),
            pltpu.VMEM((_C, _G), jnp.float32),
            pltpu.VMEM((_D, _C), jnp.float32),
            pltpu.VMEM((_D, _C), jnp.float32),
            pltpu.SemaphoreType.DMA,
            pltpu.SemaphoreType.DMA,
            pltpu.SemaphoreType.DMA,
            pltpu.SemaphoreType.DMA,
        ],
        compiler_params=pltpu.CompilerParams(needs_layout_passes=False),
    )(table4, idx)
    return jnp.transpose(out3, (2, 0, 1))


# speedup vs baseline: 1.9289x; 1.3797x over previous
import jax
import jax.numpy as jnp
from jax import lax
from jax.experimental import pallas as pl
from jax.experimental.pallas import tpu as pltpu
from jax.experimental.pallas import tpu_sc as plsc

_D = 32
_G = 128
_NI = 16384
_NJ = 50
_NW = 32
_SI = _NI // _NW
_C = 256
_NCHUNK = _NJ * (_SI // _C)


def _body(table4, idx_hbm, out3, idxblk, gidx0, gidx1, obuf0, obuf1,
          gath0, gath1, ost0, ost1, gsem0, gsem1, osem0, osem1):
    wid = lax.axis_index("s") * 2 + lax.axis_index("c")
    i0 = wid * _SI

    pltpu.sync_copy(idx_hbm.at[pl.ds(i0 * _NJ, _SI * _NJ)], idxblk)

    gidx = (gidx0, gidx1)
    obuf = (obuf0, obuf1)
    gath = (gath0, gath1)
    ost = (ost0, ost1)
    gsem = (gsem0, gsem1)
    osem = (osem0, osem1)

    riota = lax.iota(jnp.int32, 16)

    def prep_and_fire(k, b):
        j = k // 2
        h = k % 2
        jvec = jnp.full((16,), j, dtype=jnp.int32)

        @plsc.parallel_loop(0, _C // 16, unroll=4)
        def _(m):
            ivec = riota + (h * _C + 16 * m)
            raw = plsc.load_gather(idxblk, [ivec * _NJ + jvec])
            gidx[b][pl.ds(16 * m, 16)] = lax.shift_right_logical(raw, 2)
            obuf[b][pl.ds(16 * m, 16)] = (raw & 3) * _D

        for q in range(_C // 128):
            pltpu.async_copy(
                table4.at[gidx[b].at[pl.ds(q * 128, 128)]],
                gath[b].at[pl.ds(q * 128, 128)], gsem[b])

    def drain_gather(b):
        for q in range(_C // 128):
            pltpu.make_async_copy(
                table4.at[gidx[b].at[pl.ds(q * 128, 128)]],
                gath[b].at[pl.ds(q * 128, 128)], gsem[b]).wait()

    def store_dma(k, b):
        j = k // 2
        h = k % 2
        return pltpu.make_async_copy(
            ost[b], out3.at[j, :, pl.ds(i0 + h * _C, _C)], osem[b])

    def extract(b):
        @plsc.parallel_loop(0, _C // 16, unroll=2)
        def _(m):
            pvec = riota + 16 * m
            ovec = obuf[b][pl.ds(16 * m, 16)]
            for c in range(_D):
                val = plsc.load_gather(gath[b], [pvec, ovec + c])
                ost[b][c, pl.ds(16 * m, 16)] = val

    prep_and_fire(0, 0)

    def step(g, carry):
        for b in range(2):
            k = 2 * g + b
            nb = 1 - b

            @pl.when(k + 1 < _NCHUNK)
            def _():
                prep_and_fire(k + 1, nb)

            drain_gather(b)

            @pl.when(k >= 2)
            def _():
                store_dma(k - 2, b).wait()

            extract(b)

            pltpu.async_copy(
                ost[b], out3.at[k // 2, :, pl.ds(i0 + (k % 2) * _C, _C)],
                osem[b])
        return carry

    lax.fori_loop(0, _NCHUNK // 2, step, 0)

    store_dma(_NCHUNK - 2, 0).wait()
    store_dma(_NCHUNK - 1, 1).wait()


@jax.jit
def kernel(input, index):
    table4 = input.reshape(input.shape[0] * _D // _G, _G)
    idx = index.astype(jnp.int32).reshape(-1)
    mesh = plsc.VectorSubcoreMesh(core_axis_name="c", subcore_axis_name="s")
    out3 = pl.kernel(
        _body,
        mesh=mesh,
        out_type=jax.ShapeDtypeStruct((_NJ, _D, _NI), jnp.float32),
        scratch_types=[
            pltpu.VMEM((_SI * _NJ,), jnp.int32),
            pltpu.VMEM((_C,), jnp.int32),
            pltpu.VMEM((_C,), jnp.int32),
            pltpu.VMEM((_C,), jnp.int32),
            pltpu.VMEM((_C,), jnp.int32),
            pltpu.VMEM((_C, _G), jnp.float32),
            pltpu.VMEM((_C, _G), jnp.float32),
            pltpu.VMEM((_D, _C), jnp.float32),
            pltpu.VMEM((_D, _C), jnp.float32),
            pltpu.SemaphoreType.DMA,
            pltpu.SemaphoreType.DMA,
            pltpu.SemaphoreType.DMA,
            pltpu.SemaphoreType.DMA,
        ],
        compiler_params=pltpu.CompilerParams(needs_layout_passes=False),
    )(table4, idx)
    return jnp.transpose(out3, (2, 0, 1))
